# Initial kernel scaffold; baseline (speedup 1.0000x reference)
import jax
import jax.numpy as jnp
from jax.experimental import pallas as pl


def kernel(x, edge_index, W1, b1, W2, b2):
    raise NotImplementedError("write your pallas kernel here")


# speedup vs baseline: 30.8649x; 30.8649x over previous
import functools

import jax
import jax.numpy as jnp
from jax import lax
from jax.experimental import pallas as pl
from jax.experimental.pallas import tpu as pltpu
from jax.experimental.pallas import tpu_sc as plsc

N = 10000
E = 320000
D_IN = 128
D_HID = 16
D_OUT = 2

NC = 2
NS = 16
L = 16
NW = NC * NS
CHUNK = 128
CPW = -(-E // (NW * CHUNK))
EPAD = NW * CPW * CHUNK
NPAD = 10240
ROWS_PT = NPAD // NS
WB_PT = NPAD // NS

_mesh = plsc.VectorSubcoreMesh(core_axis_name="c", subcore_axis_name="s")
_sc_params = pltpu.CompilerParams(use_tc_tiling_on_sc=False)


@functools.partial(
    pl.kernel,
    out_type=jax.ShapeDtypeStruct((NC, NPAD, L), jnp.float32),
    mesh=_mesh,
    compiler_params=_sc_params,
    scratch_types=[
        pltpu.VMEM((CPW, CHUNK), jnp.int32),
        pltpu.VMEM((CHUNK, L), jnp.float32),
        pltpu.VMEM((ROWS_PT, L), jnp.float32),
        pltpu.VMEM_SHARED((NPAD, L), jnp.float32),
    ],
)
def _sc_degree(dstR, out, dst_v, ones_v, zb, acc):
    cid = lax.axis_index("c")
    sid = lax.axis_index("s")
    wid = cid * NS + sid

    def fill(i, _):
        zb[i, :] = jnp.zeros((L,), jnp.float32)
        return 0

    lax.fori_loop(0, ROWS_PT, fill, 0, unroll=False)

    def fill1(i, _):
        ones_v[i, :] = jnp.ones((L,), jnp.float32)
        return 0

    lax.fori_loop(0, CHUNK, fill1, 0, unroll=False)
    pltpu.sync_copy(zb, acc.at[pl.ds(sid * ROWS_PT, ROWS_PT)])
    pltpu.sync_copy(dstR.at[wid], dst_v)
    plsc.subcore_barrier()

    def step(j, _):
        pltpu.sync_copy(ones_v, acc.at[dst_v.at[j]], add=True)
        return 0

    lax.fori_loop(0, CPW, step, 0, unroll=False)
    plsc.subcore_barrier()
    pltpu.sync_copy(acc.at[pl.ds(sid * WB_PT, WB_PT)],
                    out.at[cid, pl.ds(sid * WB_PT, WB_PT)])


@functools.partial(
    pl.kernel,
    out_type=jax.ShapeDtypeStruct((NC, NPAD, L), jnp.float32),
    mesh=_mesh,
    compiler_params=_sc_params,
    scratch_types=[
        pltpu.VMEM((CPW, CHUNK), jnp.int32),
        pltpu.VMEM((CPW, CHUNK), jnp.int32),
        pltpu.VMEM((CHUNK, L), jnp.float32),
        pltpu.VMEM((ROWS_PT, L), jnp.float32),
        pltpu.SemaphoreType.DMA,
        pltpu.VMEM_SHARED((NPAD, L), jnp.float32),
    ],
)
def _sc_spmm(tbl, srcR, dstR, out, src_v, dst_v, rows_v, zb, sem, acc):
    cid = lax.axis_index("c")
    sid = lax.axis_index("s")
    wid = cid * NS + sid

    def fill(i, _):
        zb[i, :] = jnp.zeros((L,), jnp.float32)
        return 0

    lax.fori_loop(0, ROWS_PT, fill, 0, unroll=False)
    pltpu.sync_copy(zb, acc.at[pl.ds(sid * ROWS_PT, ROWS_PT)])
    pltpu.sync_copy(srcR.at[wid], src_v)
    pltpu.sync_copy(dstR.at[wid], dst_v)
    plsc.subcore_barrier()

    def step(j, _):
        pltpu.async_copy(tbl.at[src_v.at[j]], rows_v, sem).wait()
        pltpu.sync_copy(rows_v, acc.at[dst_v.at[j]], add=True)
        return 0

    lax.fori_loop(0, CPW, step, 0, unroll=False)
    plsc.subcore_barrier()
    pltpu.sync_copy(acc.at[pl.ds(sid * WB_PT, WB_PT)],
                    out.at[cid, pl.ds(sid * WB_PT, WB_PT)])


def _tc_h1_body(x_ref, w_ref, d0_ref, d1_ref, h_ref, dv_ref):
    deg = d0_ref[...] + d1_ref[...] + 1.0
    dinv = lax.rsqrt(deg)
    h = jnp.dot(x_ref[...], w_ref[...], preferred_element_type=jnp.float32)
    h_ref[...] = h * dinv
    dv_ref[...] = dinv


def _tc_mid_body(s0_ref, s1_ref, h_ref, dv_ref, b_ref, g_ref):
    dinv = dv_ref[...]
    seg = s0_ref[...] + s1_ref[...] + h_ref[...]
    g_ref[...] = jnp.maximum(dinv * seg + b_ref[...], 0.0) * dinv


def _tc_out_body(a0_ref, a1_ref, g_ref, dv_ref, w_ref, b_ref, o_ref):
    agg = a0_ref[...] + a1_ref[...] + g_ref[...]
    o = jnp.dot(agg, w_ref[...], preferred_element_type=jnp.float32)
    o_ref[...] = dv_ref[...][:, :D_OUT] * o + b_ref[...]


_tc_h1 = pl.pallas_call(
    _tc_h1_body,
    out_shape=(
        jax.ShapeDtypeStruct((NPAD, L), jnp.float32),
        jax.ShapeDtypeStruct((NPAD, L), jnp.float32),
    ),
)

_tc_mid = pl.pallas_call(
    _tc_mid_body,
    out_shape=jax.ShapeDtypeStruct((NPAD, L), jnp.float32),
)

_tc_out = pl.pallas_call(
    _tc_out_body,
    out_shape=jax.ShapeDtypeStruct((NPAD, D_OUT), jnp.float32),
)


def kernel(x, edge_index, W1, b1, W2, b2):
    npad_e = EPAD - E
    src = jnp.concatenate(
        [edge_index[0], jnp.zeros((npad_e,), jnp.int32)])
    dst = jnp.concatenate(
        [edge_index[1],
         N + (jnp.arange(npad_e, dtype=jnp.int32) % (NPAD - N))])
    srcR = src.reshape(NW, CPW, CHUNK)
    dstR = dst.reshape(NW, CPW, CHUNK)

    xp = jnp.zeros((NPAD, D_IN), jnp.float32).at[:N].set(x)

    degp = _sc_degree(dstR)
    h1, dinv16 = _tc_h1(xp, W1, degp[0], degp[1])
    seg1 = _sc_spmm(h1, srcR, dstR)
    g = _tc_mid(seg1[0], seg1[1], h1, dinv16, b1.reshape(1, L))
    agg = _sc_spmm(g, srcR, dstR)
    out = _tc_out(agg[0], agg[1], g, dinv16, W2, b2.reshape(1, D_OUT))
    return out[:N]


# speedup vs baseline: 34.6509x; 1.1227x over previous
import functools

import jax
import jax.numpy as jnp
from jax import lax
from jax.experimental import pallas as pl
from jax.experimental.pallas import tpu as pltpu
from jax.experimental.pallas import tpu_sc as plsc

N = 10000
E = 320000
D_IN = 128
D_HID = 16
D_OUT = 2

NC = 2
NS = 16
L = 16
NW = NC * NS
CHUNK = 128
CPW = 80
HALF = CPW // 2
EPAD = NW * CPW * CHUNK
NPAD = 10240
ROWS_PT = NPAD // NS
WB_PT = NPAD // NS

_mesh = plsc.VectorSubcoreMesh(core_axis_name="c", subcore_axis_name="s")
_sc_params = pltpu.CompilerParams(use_tc_tiling_on_sc=False)


@functools.partial(
    pl.kernel,
    out_type=jax.ShapeDtypeStruct((NC, NPAD, L), jnp.float32),
    mesh=_mesh,
    compiler_params=_sc_params,
    scratch_types=[
        pltpu.VMEM((CPW, CHUNK), jnp.int32),
        pltpu.VMEM((CHUNK, L), jnp.float32),
        pltpu.VMEM((ROWS_PT, L), jnp.float32),
        pltpu.VMEM_SHARED((NPAD, L), jnp.float32),
    ],
)
def _sc_degree(dstR, out, dst_v, ones_v, zb, acc):
    cid = lax.axis_index("c")
    sid = lax.axis_index("s")
    wid = cid * NS + sid

    def fill(i, _):
        zb[i, :] = jnp.zeros((L,), jnp.float32)
        return 0

    lax.fori_loop(0, ROWS_PT, fill, 0, unroll=False)

    def fill1(i, _):
        ones_v[i, :] = jnp.ones((L,), jnp.float32)
        return 0

    lax.fori_loop(0, CHUNK, fill1, 0, unroll=False)
    pltpu.sync_copy(zb, acc.at[pl.ds(sid * ROWS_PT, ROWS_PT)])
    pltpu.sync_copy(dstR.at[wid], dst_v)
    plsc.subcore_barrier()

    def step(j, _):
        pltpu.sync_copy(ones_v, acc.at[dst_v.at[j]], add=True)
        return 0

    lax.fori_loop(0, CPW, step, 0, unroll=False)
    plsc.subcore_barrier()
    pltpu.sync_copy(acc.at[pl.ds(sid * WB_PT, WB_PT)],
                    out.at[cid, pl.ds(sid * WB_PT, WB_PT)])


@functools.partial(
    pl.kernel,
    out_type=jax.ShapeDtypeStruct((NC, NPAD, L), jnp.float32),
    mesh=_mesh,
    compiler_params=_sc_params,
    scratch_types=[
        pltpu.VMEM((CPW, CHUNK), jnp.int32),
        pltpu.VMEM((CPW, CHUNK), jnp.int32),
        pltpu.VMEM((CHUNK, L), jnp.float32),
        pltpu.VMEM((CHUNK, L), jnp.float32),
        pltpu.VMEM((ROWS_PT, L), jnp.float32),
        pltpu.SemaphoreType.DMA,
        pltpu.SemaphoreType.DMA,
        pltpu.VMEM_SHARED((NPAD, L), jnp.float32),
    ],
)
def _sc_spmm(tbl, srcR, dstR, out, src_v, dst_v, rows_a, rows_b, zb,
             sem_a, sem_b, acc):
    cid = lax.axis_index("c")
    sid = lax.axis_index("s")
    wid = cid * NS + sid

    def fill(i, _):
        zb[i, :] = jnp.zeros((L,), jnp.float32)
        return 0

    lax.fori_loop(0, ROWS_PT, fill, 0, unroll=False)
    pltpu.sync_copy(zb, acc.at[pl.ds(sid * ROWS_PT, ROWS_PT)])
    pltpu.sync_copy(srcR.at[wid], src_v)
    pltpu.sync_copy(dstR.at[wid], dst_v)
    plsc.subcore_barrier()

    pltpu.async_copy(tbl.at[src_v.at[0]], rows_a, sem_a)

    def step(i, _):
        j = 2 * i
        pltpu.async_copy(tbl.at[src_v.at[j + 1]], rows_b, sem_b)
        pltpu.make_async_copy(tbl.at[src_v.at[j]], rows_a, sem_a).wait()
        pltpu.sync_copy(rows_a, acc.at[dst_v.at[j]], add=True)

        @pl.when(i + 1 < HALF)
        def _():
            pltpu.async_copy(tbl.at[src_v.at[j + 2]], rows_a, sem_a)

        pltpu.make_async_copy(tbl.at[src_v.at[j + 1]], rows_b, sem_b).wait()
        pltpu.sync_copy(rows_b, acc.at[dst_v.at[j + 1]], add=True)
        return 0

    lax.fori_loop(0, HALF, step, 0, unroll=False)
    plsc.subcore_barrier()
    pltpu.sync_copy(acc.at[pl.ds(sid * WB_PT, WB_PT)],
                    out.at[cid, pl.ds(sid * WB_PT, WB_PT)])


def _tc_h1_body(x_ref, w_ref, d0_ref, d1_ref, h_ref, dv_ref):
    deg = d0_ref[...] + d1_ref[...] + 1.0
    dinv = lax.rsqrt(deg)
    h = jnp.dot(x_ref[...], w_ref[...], preferred_element_type=jnp.float32)
    h_ref[...] = h * dinv
    dv_ref[...] = dinv


def _tc_mid_body(s0_ref, s1_ref, h_ref, dv_ref, b_ref, g_ref):
    dinv = dv_ref[...]
    seg = s0_ref[...] + s1_ref[...] + h_ref[...]
    g_ref[...] = jnp.maximum(dinv * seg + b_ref[...], 0.0) * dinv


def _tc_out_body(a0_ref, a1_ref, g_ref, dv_ref, w_ref, b_ref, o_ref):
    agg = a0_ref[...] + a1_ref[...] + g_ref[...]
    o = jnp.dot(agg, w_ref[...], preferred_element_type=jnp.float32)
    o_ref[...] = dv_ref[...][:, :D_OUT] * o + b_ref[...]


_tc_h1 = pl.pallas_call(
    _tc_h1_body,
    out_shape=(
        jax.ShapeDtypeStruct((NPAD, L), jnp.float32),
        jax.ShapeDtypeStruct((NPAD, L), jnp.float32),
    ),
)

_tc_mid = pl.pallas_call(
    _tc_mid_body,
    out_shape=jax.ShapeDtypeStruct((NPAD, L), jnp.float32),
)

_tc_out = pl.pallas_call(
    _tc_out_body,
    out_shape=jax.ShapeDtypeStruct((NPAD, D_OUT), jnp.float32),
)


def kernel(x, edge_index, W1, b1, W2, b2):
    npad_e = EPAD - E
    src = jnp.concatenate(
        [edge_index[0], jnp.zeros((npad_e,), jnp.int32)])
    dst = jnp.concatenate(
        [edge_index[1],
         N + (jnp.arange(npad_e, dtype=jnp.int32) % (NPAD - N))])
    srcR = src.reshape(NW, CPW, CHUNK)
    dstR = dst.reshape(NW, CPW, CHUNK)

    xp = jnp.zeros((NPAD, D_IN), jnp.float32).at[:N].set(x)

    degp = _sc_degree(dstR)
    h1, dinv16 = _tc_h1(xp, W1, degp[0], degp[1])
    seg1 = _sc_spmm(h1, srcR, dstR)
    g = _tc_mid(seg1[0], seg1[1], h1, dinv16, b1.reshape(1, L))
    agg = _sc_spmm(g, srcR, dstR)
    out = _tc_out(agg[0], agg[1], g, dinv16, W2, b2.reshape(1, D_OUT))
    return out[:N]


# speedup vs baseline: 35.3968x; 1.0215x over previous
import functools

import jax
import jax.numpy as jnp
from jax import lax
from jax.experimental import pallas as pl
from jax.experimental.pallas import tpu as pltpu
from jax.experimental.pallas import tpu_sc as plsc

N = 10000
E = 320000
D_IN = 128
D_HID = 16
D_OUT = 2

NC = 2
NS = 16
L = 16
NW = NC * NS
CHUNK = 128
CPW = 80
HALF = CPW // 2
EPAD = NW * CPW * CHUNK
NPAD = 10240
ROWS_PT = NPAD // NS
WB_PT = NPAD // NS

_mesh = plsc.VectorSubcoreMesh(core_axis_name="c", subcore_axis_name="s")
_sc_params = pltpu.CompilerParams(use_tc_tiling_on_sc=False)


@functools.partial(
    pl.kernel,
    out_type=jax.ShapeDtypeStruct((NC, NPAD, L), jnp.float32),
    mesh=_mesh,
    compiler_params=_sc_params,
    scratch_types=[
        pltpu.VMEM((CPW, CHUNK), jnp.int32),
        pltpu.VMEM((CHUNK, L), jnp.float32),
        pltpu.VMEM((ROWS_PT, L), jnp.float32),
        pltpu.VMEM_SHARED((NPAD, L), jnp.float32),
    ],
)
def _sc_degree(dstR, out, dst_v, ones_v, zb, acc):
    cid = lax.axis_index("c")
    sid = lax.axis_index("s")
    wid = cid * NS + sid

    def fill(i, _):
        zb[i, :] = jnp.zeros((L,), jnp.float32)
        return 0

    lax.fori_loop(0, ROWS_PT, fill, 0, unroll=False)

    def fill1(i, _):
        ones_v[i, :] = jnp.ones((L,), jnp.float32)
        return 0

    lax.fori_loop(0, CHUNK, fill1, 0, unroll=False)
    pltpu.sync_copy(zb, acc.at[pl.ds(sid * ROWS_PT, ROWS_PT)])
    pltpu.sync_copy(dstR.at[wid], dst_v)
    plsc.subcore_barrier()

    def step(j, _):
        pltpu.sync_copy(ones_v, acc.at[dst_v.at[j]], add=True)
        return 0

    lax.fori_loop(0, CPW, step, 0, unroll=False)
    plsc.subcore_barrier()
    pltpu.sync_copy(acc.at[pl.ds(sid * WB_PT, WB_PT)],
                    out.at[cid, pl.ds(sid * WB_PT, WB_PT)])


@functools.partial(
    pl.kernel,
    out_type=jax.ShapeDtypeStruct((NC, NPAD, L), jnp.float32),
    mesh=_mesh,
    compiler_params=_sc_params,
    scratch_types=[
        pltpu.VMEM((CPW, CHUNK), jnp.int32),
        pltpu.VMEM((CPW, CHUNK), jnp.int32),
        pltpu.VMEM((CHUNK, L), jnp.float32),
        pltpu.VMEM((CHUNK, L), jnp.float32),
        pltpu.VMEM((CHUNK, L), jnp.float32),
        pltpu.VMEM((CHUNK, L), jnp.float32),
        pltpu.VMEM((ROWS_PT, L), jnp.float32),
        pltpu.SemaphoreType.DMA,
        pltpu.SemaphoreType.DMA,
        pltpu.SemaphoreType.DMA,
        pltpu.SemaphoreType.DMA,
        pltpu.VMEM_SHARED((NPAD, L), jnp.float32),
    ],
)
def _sc_spmm(tbl, srcR, dstR, out, src_v, dst_v, r0, r1, r2, r3, zb,
             s0, s1, s2, s3, acc):
    cid = lax.axis_index("c")
    sid = lax.axis_index("s")
    wid = cid * NS + sid

    def fill(i, _):
        zb[i, :] = jnp.zeros((L,), jnp.float32)
        return 0

    lax.fori_loop(0, ROWS_PT, fill, 0, unroll=False)
    pltpu.sync_copy(zb, acc.at[pl.ds(sid * ROWS_PT, ROWS_PT)])
    pltpu.sync_copy(srcR.at[wid], src_v)
    pltpu.sync_copy(dstR.at[wid], dst_v)
    plsc.subcore_barrier()

    rings = (r0, r1, r2, r3)
    sems = (s0, s1, s2, s3)
    for b in range(3):
        pltpu.async_copy(tbl.at[src_v.at[b]], rings[b], sems[b])

    def group(i, _):
        for b in range(4):
            j = 4 * i + b
            pltpu.make_async_copy(tbl.at[src_v.at[j]], rings[b], sems[b]).wait()
            pltpu.sync_copy(rings[b], acc.at[dst_v.at[j]], add=True)
            nb = (b + 3) % 4

            @pl.when(j + 3 < CPW)
            def _():
                pltpu.async_copy(tbl.at[src_v.at[j + 3]], rings[nb], sems[nb])
        return 0

    lax.fori_loop(0, CPW // 4, group, 0, unroll=False)
    plsc.subcore_barrier()
    pltpu.sync_copy(acc.at[pl.ds(sid * WB_PT, WB_PT)],
                    out.at[cid, pl.ds(sid * WB_PT, WB_PT)])


def _tc_h1_body(x_ref, w_ref, d0_ref, d1_ref, h_ref, dv_ref):
    deg = d0_ref[...] + d1_ref[...] + 1.0
    dinv = lax.rsqrt(deg)
    h = jnp.dot(x_ref[...], w_ref[...], preferred_element_type=jnp.float32)
    h_ref[...] = h * dinv
    dv_ref[...] = dinv


def _tc_mid_body(s0_ref, s1_ref, h_ref, dv_ref, b_ref, g_ref):
    dinv = dv_ref[...]
    seg = s0_ref[...] + s1_ref[...] + h_ref[...]
    g_ref[...] = jnp.maximum(dinv * seg + b_ref[...], 0.0) * dinv


def _tc_out_body(a0_ref, a1_ref, g_ref, dv_ref, w_ref, b_ref, o_ref):
    agg = a0_ref[...] + a1_ref[...] + g_ref[...]
    o = jnp.dot(agg, w_ref[...], preferred_element_type=jnp.float32)
    o_ref[...] = dv_ref[...][:, :D_OUT] * o + b_ref[...]


_tc_h1 = pl.pallas_call(
    _tc_h1_body,
    out_shape=(
        jax.ShapeDtypeStruct((NPAD, L), jnp.float32),
        jax.ShapeDtypeStruct((NPAD, L), jnp.float32),
    ),
)

_tc_mid = pl.pallas_call(
    _tc_mid_body,
    out_shape=jax.ShapeDtypeStruct((NPAD, L), jnp.float32),
)

_tc_out = pl.pallas_call(
    _tc_out_body,
    out_shape=jax.ShapeDtypeStruct((NPAD, D_OUT), jnp.float32),
)


def kernel(x, edge_index, W1, b1, W2, b2):
    npad_e = EPAD - E
    src = jnp.concatenate(
        [edge_index[0], jnp.zeros((npad_e,), jnp.int32)])
    dst = jnp.concatenate(
        [edge_index[1],
         N + (jnp.arange(npad_e, dtype=jnp.int32) % (NPAD - N))])
    srcR = src.reshape(NW, CPW, CHUNK)
    dstR = dst.reshape(NW, CPW, CHUNK)

    xp = jnp.zeros((NPAD, D_IN), jnp.float32).at[:N].set(x)

    degp = _sc_degree(dstR)
    h1, dinv16 = _tc_h1(xp, W1, degp[0], degp[1])
    seg1 = _sc_spmm(h1, srcR, dstR)
    g = _tc_mid(seg1[0], seg1[1], h1, dinv16, b1.reshape(1, L))
    agg = _sc_spmm(g, srcR, dstR)
    out = _tc_out(agg[0], agg[1], g, dinv16, W2, b2.reshape(1, D_OUT))
    return out[:N]


# speedup vs baseline: 41.8416x; 1.1821x over previous
import functools

import jax
import jax.numpy as jnp
from jax import lax
from jax.experimental import pallas as pl
from jax.experimental.pallas import tpu as pltpu
from jax.experimental.pallas import tpu_sc as plsc

N = 10000
E = 320000
D_IN = 128
D_HID = 16
D_OUT = 2

NC = 2
NS = 16
L = 16
NW = NC * NS
CHUNK = 128
CPW = 80
HALF = CPW // 2
EPAD = NW * CPW * CHUNK
NPAD = 10240
ROWS_PT = NPAD // NS
WB_PT = NPAD // NS

_mesh = plsc.VectorSubcoreMesh(core_axis_name="c", subcore_axis_name="s")
_sc_params = pltpu.CompilerParams(use_tc_tiling_on_sc=False)


@functools.partial(
    pl.kernel,
    out_type=jax.ShapeDtypeStruct((NC, NPAD, L), jnp.float32),
    mesh=_mesh,
    compiler_params=_sc_params,
    scratch_types=[
        pltpu.VMEM((CPW, CHUNK), jnp.int32),
        pltpu.VMEM((CHUNK, L), jnp.float32),
        pltpu.VMEM((ROWS_PT, L), jnp.float32),
        pltpu.VMEM_SHARED((NPAD, L), jnp.float32),
    ],
)
def _sc_degree(dstR, out, dst_v, ones_v, zb, acc):
    cid = lax.axis_index("c")
    sid = lax.axis_index("s")
    wid = cid * NS + sid

    def fill(i, _):
        zb[i, :] = jnp.zeros((L,), jnp.float32)
        return 0

    lax.fori_loop(0, ROWS_PT, fill, 0, unroll=False)

    def fill1(i, _):
        ones_v[i, :] = jnp.ones((L,), jnp.float32)
        return 0

    lax.fori_loop(0, CHUNK, fill1, 0, unroll=False)
    pltpu.sync_copy(zb, acc.at[pl.ds(sid * ROWS_PT, ROWS_PT)])
    pltpu.sync_copy(dstR.at[wid], dst_v)
    plsc.subcore_barrier()

    def step(j, _):
        pltpu.sync_copy(ones_v, acc.at[dst_v.at[j]], add=True)
        return 0

    lax.fori_loop(0, CPW, step, 0, unroll=False)
    plsc.subcore_barrier()
    pltpu.sync_copy(acc.at[pl.ds(sid * WB_PT, WB_PT)],
                    out.at[cid, pl.ds(sid * WB_PT, WB_PT)])


@functools.partial(
    pl.kernel,
    out_type=jax.ShapeDtypeStruct((NC, NPAD, L), jnp.float32),
    mesh=_mesh,
    compiler_params=_sc_params,
    scratch_types=[
        pltpu.VMEM((CPW, CHUNK), jnp.int32),
        pltpu.VMEM((CPW, CHUNK), jnp.int32),
        pltpu.VMEM((CHUNK, L), jnp.float32),
        pltpu.VMEM((CHUNK, L), jnp.float32),
        pltpu.VMEM((CHUNK, L), jnp.float32),
        pltpu.VMEM((CHUNK, L), jnp.float32),
        pltpu.VMEM((ROWS_PT, L), jnp.float32),
        pltpu.SemaphoreType.DMA,
        pltpu.SemaphoreType.DMA,
        pltpu.SemaphoreType.DMA,
        pltpu.SemaphoreType.DMA,
        pltpu.VMEM_SHARED((NPAD, L), jnp.float32),
    ],
)
def _sc_spmm(tbl, srcR, dstR, out, src_v, dst_v, r0, r1, r2, r3, zb,
             s0, s1, s2, s3, acc):
    cid = lax.axis_index("c")
    sid = lax.axis_index("s")
    wid = cid * NS + sid

    def fill(i, _):
        zb[i, :] = jnp.zeros((L,), jnp.float32)
        return 0

    lax.fori_loop(0, ROWS_PT, fill, 0, unroll=False)
    pltpu.sync_copy(zb, acc.at[pl.ds(sid * ROWS_PT, ROWS_PT)])
    pltpu.sync_copy(srcR.at[wid], src_v)
    pltpu.sync_copy(dstR.at[wid], dst_v)
    plsc.subcore_barrier()

    rings = (r0, r1, r2, r3)
    sems = (s0, s1, s2, s3)
    for b in range(3):
        pltpu.async_copy(tbl.at[src_v.at[b]], rings[b], sems[b])

    def group(i, _):
        for b in range(4):
            j = 4 * i + b
            pltpu.make_async_copy(tbl.at[src_v.at[j]], rings[b], sems[b]).wait()
            pltpu.sync_copy(rings[b], acc.at[dst_v.at[j]], add=True)
            nb = (b + 3) % 4

            @pl.when(j + 3 < CPW)
            def _():
                pltpu.async_copy(tbl.at[src_v.at[j + 3]], rings[nb], sems[nb])
        return 0

    lax.fori_loop(0, CPW // 4, group, 0, unroll=False)
    plsc.subcore_barrier()
    pltpu.sync_copy(acc.at[pl.ds(sid * WB_PT, WB_PT)],
                    out.at[cid, pl.ds(sid * WB_PT, WB_PT)])


def _tc_h1_body(x_ref, w_ref, d_ref, h_ref, dv_ref):
    deg = d_ref[0] + d_ref[1] + 1.0
    dinv = lax.rsqrt(deg)
    h = jnp.dot(x_ref[...], w_ref[...], preferred_element_type=jnp.float32)
    h_ref[:N, :] = h * dinv[:N]
    h_ref[N:, :] = jnp.zeros((NPAD - N, L), jnp.float32)
    dv_ref[...] = dinv


def _tc_mid_body(s_ref, h_ref, dv_ref, b_ref, g_ref):
    dinv = dv_ref[...]
    seg = s_ref[0] + s_ref[1] + h_ref[...]
    g_ref[...] = jnp.maximum(dinv * seg + b_ref[...], 0.0) * dinv


def _tc_out_body(a_ref, g_ref, dv_ref, w_ref, b_ref, o_ref):
    agg = a_ref[0, :N, :] + a_ref[1, :N, :] + g_ref[:N, :]
    o = jnp.dot(agg, w_ref[...], preferred_element_type=jnp.float32)
    o_ref[...] = dv_ref[:N, :D_OUT] * o + b_ref[...]


_tc_h1 = pl.pallas_call(
    _tc_h1_body,
    out_shape=(
        jax.ShapeDtypeStruct((NPAD, L), jnp.float32),
        jax.ShapeDtypeStruct((NPAD, L), jnp.float32),
    ),
)

_tc_mid = pl.pallas_call(
    _tc_mid_body,
    out_shape=jax.ShapeDtypeStruct((NPAD, L), jnp.float32),
)

_tc_out = pl.pallas_call(
    _tc_out_body,
    out_shape=jax.ShapeDtypeStruct((N, D_OUT), jnp.float32),
)


def kernel(x, edge_index, W1, b1, W2, b2):
    npad_e = EPAD - E
    src = jnp.concatenate(
        [edge_index[0], jnp.zeros((npad_e,), jnp.int32)])
    dst = jnp.concatenate(
        [edge_index[1],
         N + (jnp.arange(npad_e, dtype=jnp.int32) % (NPAD - N))])
    srcR = src.reshape(NW, CPW, CHUNK)
    dstR = dst.reshape(NW, CPW, CHUNK)

    degp = _sc_degree(dstR)
    h1, dinv16 = _tc_h1(x, W1, degp)
    seg1 = _sc_spmm(h1, srcR, dstR)
    g = _tc_mid(seg1, h1, dinv16, b1.reshape(1, L))
    agg = _sc_spmm(g, srcR, dstR)
    return _tc_out(agg, g, dinv16, W2, b2.reshape(1, D_OUT))


# speedup vs baseline: 51.2844x; 1.2257x over previous
import functools

import jax
import jax.numpy as jnp
from jax import lax
from jax.experimental import pallas as pl
from jax.experimental.pallas import tpu as pltpu
from jax.experimental.pallas import tpu_sc as plsc

N = 10000
E = 320000
D_IN = 128
D_HID = 16
D_OUT = 2

NC = 2
NS = 16
L = 16
NW = NC * NS
CHUNK = 125
CPW = 80
ROWS_PT = N // NS

_mesh = plsc.VectorSubcoreMesh(core_axis_name="c", subcore_axis_name="s")
_sc_params = pltpu.CompilerParams(use_tc_tiling_on_sc=False)


@functools.partial(
    pl.kernel,
    out_type=jax.ShapeDtypeStruct((NC, N, L), jnp.float32),
    mesh=_mesh,
    compiler_params=_sc_params,
    scratch_types=[
        pltpu.VMEM((CPW, CHUNK), jnp.int32),
        pltpu.VMEM((CHUNK, L), jnp.float32),
        pltpu.VMEM((ROWS_PT, L), jnp.float32),
        pltpu.VMEM_SHARED((N, L), jnp.float32),
    ],
)
def _sc_degree(dstR, out, dst_v, ones_v, zb, acc):
    cid = lax.axis_index("c")
    sid = lax.axis_index("s")
    wid = cid * NS + sid

    def fill(i, _):
        zb[i, :] = jnp.zeros((L,), jnp.float32)
        return 0

    lax.fori_loop(0, ROWS_PT, fill, 0, unroll=False)

    def fill1(i, _):
        ones_v[i, :] = jnp.ones((L,), jnp.float32)
        return 0

    lax.fori_loop(0, CHUNK, fill1, 0, unroll=False)
    pltpu.sync_copy(zb, acc.at[pl.ds(sid * ROWS_PT, ROWS_PT)])
    pltpu.sync_copy(dstR.at[wid], dst_v)
    plsc.subcore_barrier()

    def step(j, _):
        pltpu.sync_copy(ones_v, acc.at[dst_v.at[j]], add=True)
        return 0

    lax.fori_loop(0, CPW, step, 0, unroll=False)
    plsc.subcore_barrier()
    pltpu.sync_copy(acc.at[pl.ds(sid * ROWS_PT, ROWS_PT)],
                    out.at[cid, pl.ds(sid * ROWS_PT, ROWS_PT)])


@functools.partial(
    pl.kernel,
    out_type=jax.ShapeDtypeStruct((NC, N, L), jnp.float32),
    mesh=_mesh,
    compiler_params=_sc_params,
    scratch_types=[
        pltpu.VMEM((CPW, CHUNK), jnp.int32),
        pltpu.VMEM((CPW, CHUNK), jnp.int32),
        pltpu.VMEM((CHUNK, L), jnp.float32),
        pltpu.VMEM((CHUNK, L), jnp.float32),
        pltpu.VMEM((CHUNK, L), jnp.float32),
        pltpu.VMEM((CHUNK, L), jnp.float32),
        pltpu.VMEM((ROWS_PT, L), jnp.float32),
        pltpu.SemaphoreType.DMA,
        pltpu.SemaphoreType.DMA,
        pltpu.SemaphoreType.DMA,
        pltpu.SemaphoreType.DMA,
        pltpu.VMEM_SHARED((N, L), jnp.float32),
    ],
)
def _sc_spmm(tbl, srcR, dstR, out, src_v, dst_v, r0, r1, r2, r3, zb,
             s0, s1, s2, s3, acc):
    cid = lax.axis_index("c")
    sid = lax.axis_index("s")
    wid = cid * NS + sid

    def fill(i, _):
        zb[i, :] = jnp.zeros((L,), jnp.float32)
        return 0

    lax.fori_loop(0, ROWS_PT, fill, 0, unroll=False)
    pltpu.sync_copy(zb, acc.at[pl.ds(sid * ROWS_PT, ROWS_PT)])
    pltpu.sync_copy(srcR.at[wid], src_v)
    pltpu.sync_copy(dstR.at[wid], dst_v)
    plsc.subcore_barrier()

    rings = (r0, r1, r2, r3)
    sems = (s0, s1, s2, s3)
    for b in range(3):
        pltpu.async_copy(tbl.at[src_v.at[b]], rings[b], sems[b])

    def group(i, _):
        for b in range(4):
            j = 4 * i + b
            pltpu.make_async_copy(tbl.at[src_v.at[j]], rings[b], sems[b]).wait()
            pltpu.sync_copy(rings[b], acc.at[dst_v.at[j]], add=True)
            nb = (b + 3) % 4

            @pl.when(j + 3 < CPW)
            def _():
                pltpu.async_copy(tbl.at[src_v.at[j + 3]], rings[nb], sems[nb])
        return 0

    lax.fori_loop(0, CPW // 4, group, 0, unroll=False)
    plsc.subcore_barrier()
    pltpu.sync_copy(acc.at[pl.ds(sid * ROWS_PT, ROWS_PT)],
                    out.at[cid, pl.ds(sid * ROWS_PT, ROWS_PT)])


def _tc_h1_body(x_ref, w_ref, d_ref, h_ref, dv_ref):
    deg = d_ref[0] + d_ref[1] + 1.0
    dinv = lax.rsqrt(deg)
    h = jnp.dot(x_ref[...], w_ref[...], preferred_element_type=jnp.float32)
    h_ref[...] = h * dinv
    dv_ref[...] = dinv


def _tc_mid_body(s_ref, h_ref, dv_ref, b_ref, g_ref):
    dinv = dv_ref[...]
    seg = s_ref[0] + s_ref[1] + h_ref[...]
    g_ref[...] = jnp.maximum(dinv * seg + b_ref[...], 0.0) * dinv


def _tc_out_body(a_ref, g_ref, dv_ref, w_ref, b_ref, o_ref):
    agg = a_ref[0] + a_ref[1] + g_ref[...]
    o = jnp.dot(agg, w_ref[...], preferred_element_type=jnp.float32)
    o_ref[...] = dv_ref[:, :D_OUT] * o + b_ref[...]


_tc_h1 = pl.pallas_call(
    _tc_h1_body,
    out_shape=(
        jax.ShapeDtypeStruct((N, L), jnp.float32),
        jax.ShapeDtypeStruct((N, L), jnp.float32),
    ),
)

_tc_mid = pl.pallas_call(
    _tc_mid_body,
    out_shape=jax.ShapeDtypeStruct((N, L), jnp.float32),
)

_tc_out = pl.pallas_call(
    _tc_out_body,
    out_shape=jax.ShapeDtypeStruct((N, D_OUT), jnp.float32),
)


def kernel(x, edge_index, W1, b1, W2, b2):
    srcR = edge_index[0].reshape(NW, CPW, CHUNK)
    dstR = edge_index[1].reshape(NW, CPW, CHUNK)

    degp = _sc_degree(dstR)
    h1, dinv16 = _tc_h1(x, W1, degp)
    seg1 = _sc_spmm(h1, srcR, dstR)
    g = _tc_mid(seg1, h1, dinv16, b1.reshape(1, L))
    agg = _sc_spmm(g, srcR, dstR)
    return _tc_out(agg, g, dinv16, W2, b2.reshape(1, D_OUT))


# speedup vs baseline: 52.8171x; 1.0299x over previous
import functools

import jax
import jax.numpy as jnp
from jax import lax
from jax.experimental import pallas as pl
from jax.experimental.pallas import tpu as pltpu
from jax.experimental.pallas import tpu_sc as plsc

N = 10000
E = 320000
D_IN = 128
D_HID = 16
D_OUT = 2

NC = 2
NS = 16
L = 16
NW = NC * NS
CHUNK = 125
CPW = 80
ROWS_PT = N // NS

_mesh = plsc.VectorSubcoreMesh(core_axis_name="c", subcore_axis_name="s")
_sc_params = pltpu.CompilerParams(use_tc_tiling_on_sc=False)


@functools.partial(
    pl.kernel,
    out_type=jax.ShapeDtypeStruct((NC, N, L), jnp.float32),
    mesh=_mesh,
    compiler_params=_sc_params,
    scratch_types=[
        pltpu.VMEM((CPW, CHUNK), jnp.int32),
        pltpu.VMEM((CHUNK, L), jnp.float32),
        pltpu.VMEM((ROWS_PT, L), jnp.float32),
        pltpu.VMEM_SHARED((N, L), jnp.float32),
    ],
)
def _sc_degree(dstR, out, dst_v, ones_v, zb, acc):
    cid = lax.axis_index("c")
    sid = lax.axis_index("s")
    wid = cid * NS + sid

    def fill(i, _):
        zb[i, :] = jnp.zeros((L,), jnp.float32)
        return 0

    lax.fori_loop(0, ROWS_PT, fill, 0, unroll=False)

    def fill1(i, _):
        ones_v[i, :] = jnp.ones((L,), jnp.float32)
        return 0

    lax.fori_loop(0, CHUNK, fill1, 0, unroll=False)
    pltpu.sync_copy(zb, acc.at[pl.ds(sid * ROWS_PT, ROWS_PT)])
    pltpu.sync_copy(dstR.at[wid], dst_v)
    plsc.subcore_barrier()

    def step(j, _):
        pltpu.sync_copy(ones_v, acc.at[dst_v.at[j]], add=True)
        return 0

    lax.fori_loop(0, CPW, step, 0, unroll=False)
    plsc.subcore_barrier()
    pltpu.sync_copy(acc.at[pl.ds(sid * ROWS_PT, ROWS_PT)],
                    out.at[cid, pl.ds(sid * ROWS_PT, ROWS_PT)])


@functools.partial(
    pl.kernel,
    out_type=jax.ShapeDtypeStruct((NC, N, L), jnp.float32),
    mesh=_mesh,
    compiler_params=_sc_params,
    scratch_types=[
        pltpu.VMEM((CPW, CHUNK), jnp.int32),
        pltpu.VMEM((CPW, CHUNK), jnp.int32),
        pltpu.VMEM((CHUNK, L), jnp.float32),
        pltpu.VMEM((CHUNK, L), jnp.float32),
        pltpu.VMEM((CHUNK, L), jnp.float32),
        pltpu.VMEM((CHUNK, L), jnp.float32),
        pltpu.VMEM((ROWS_PT, L), jnp.float32),
        pltpu.SemaphoreType.DMA,
        pltpu.SemaphoreType.DMA,
        pltpu.SemaphoreType.DMA,
        pltpu.SemaphoreType.DMA,
        pltpu.SemaphoreType.DMA,
        pltpu.SemaphoreType.DMA,
        pltpu.SemaphoreType.DMA,
        pltpu.SemaphoreType.DMA,
        pltpu.VMEM_SHARED((N, L), jnp.float32),
    ],
)
def _sc_spmm(tbl, srcR, dstR, out, src_v, dst_v, r0, r1, r2, r3, zb,
             s0, s1, s2, s3, t0, t1, t2, t3, acc):
    cid = lax.axis_index("c")
    sid = lax.axis_index("s")
    wid = cid * NS + sid

    def fill(i, _):
        zb[i, :] = jnp.zeros((L,), jnp.float32)
        return 0

    lax.fori_loop(0, ROWS_PT, fill, 0, unroll=False)
    pltpu.sync_copy(zb, acc.at[pl.ds(sid * ROWS_PT, ROWS_PT)])
    pltpu.sync_copy(srcR.at[wid], src_v)
    pltpu.sync_copy(dstR.at[wid], dst_v)
    plsc.subcore_barrier()

    rings = (r0, r1, r2, r3)
    gsems = (s0, s1, s2, s3)
    ssems = (t0, t1, t2, t3)
    for b in range(3):
        pltpu.async_copy(tbl.at[src_v.at[b]], rings[b], gsems[b])

    def group(i, _):
        for b in range(4):
            j = 4 * i + b
            pltpu.make_async_copy(tbl.at[src_v.at[j]], rings[b], gsems[b]).wait()
            pltpu.async_copy(rings[b], acc.at[dst_v.at[j]], ssems[b], add=True)
            nb = (b + 3) % 4

            @pl.when(j + 3 < CPW)
            def _():
                @pl.when(j >= 1)
                def _():
                    pltpu.make_async_copy(
                        rings[nb], acc.at[dst_v.at[j - 1]], ssems[nb]).wait()
                pltpu.async_copy(tbl.at[src_v.at[j + 3]], rings[nb], gsems[nb])
        return 0

    lax.fori_loop(0, CPW // 4, group, 0, unroll=False)
    for b in range(4):
        j = CPW - 4 + b
        pltpu.make_async_copy(rings[j % 4], acc.at[dst_v.at[j]],
                              ssems[j % 4]).wait()
    plsc.subcore_barrier()
    pltpu.sync_copy(acc.at[pl.ds(sid * ROWS_PT, ROWS_PT)],
                    out.at[cid, pl.ds(sid * ROWS_PT, ROWS_PT)])


def _tc_h1_body(x_ref, w_ref, d_ref, h_ref, dv_ref):
    deg = d_ref[0] + d_ref[1] + 1.0
    dinv = lax.rsqrt(deg)
    h = jnp.dot(x_ref[...], w_ref[...], preferred_element_type=jnp.float32)
    h_ref[...] = h * dinv
    dv_ref[...] = dinv


def _tc_mid_body(s_ref, h_ref, dv_ref, b_ref, g_ref):
    dinv = dv_ref[...]
    seg = s_ref[0] + s_ref[1] + h_ref[...]
    g_ref[...] = jnp.maximum(dinv * seg + b_ref[...], 0.0) * dinv


def _tc_out_body(a_ref, g_ref, dv_ref, w_ref, b_ref, o_ref):
    agg = a_ref[0] + a_ref[1] + g_ref[...]
    o = jnp.dot(agg, w_ref[...], preferred_element_type=jnp.float32)
    o_ref[...] = dv_ref[:, :D_OUT] * o + b_ref[...]


_tc_h1 = pl.pallas_call(
    _tc_h1_body,
    out_shape=(
        jax.ShapeDtypeStruct((N, L), jnp.float32),
        jax.ShapeDtypeStruct((N, L), jnp.float32),
    ),
)

_tc_mid = pl.pallas_call(
    _tc_mid_body,
    out_shape=jax.ShapeDtypeStruct((N, L), jnp.float32),
)

_tc_out = pl.pallas_call(
    _tc_out_body,
    out_shape=jax.ShapeDtypeStruct((N, D_OUT), jnp.float32),
)


def kernel(x, edge_index, W1, b1, W2, b2):
    srcR = edge_index[0].reshape(NW, CPW, CHUNK)
    dstR = edge_index[1].reshape(NW, CPW, CHUNK)

    degp = _sc_degree(dstR)
    h1, dinv16 = _tc_h1(x, W1, degp)
    seg1 = _sc_spmm(h1, srcR, dstR)
    g = _tc_mid(seg1, h1, dinv16, b1.reshape(1, L))
    agg = _sc_spmm(g, srcR, dstR)
    return _tc_out(agg, g, dinv16, W2, b2.reshape(1, D_OUT))


# speedup vs baseline: 54.4387x; 1.0307x over previous
import functools

import jax
import jax.numpy as jnp
from jax import lax
from jax.experimental import pallas as pl
from jax.experimental.pallas import tpu as pltpu
from jax.experimental.pallas import tpu_sc as plsc

N = 10000
E = 320000
D_IN = 128
D_HID = 16
D_OUT = 2

NC = 2
NS = 16
L = 16
NW = NC * NS
CHUNK = 125
CPW = 80
ROWS_PT = N // NS

_mesh = plsc.VectorSubcoreMesh(core_axis_name="c", subcore_axis_name="s")
_sc_params = pltpu.CompilerParams(use_tc_tiling_on_sc=False)


@functools.partial(
    pl.kernel,
    out_type=jax.ShapeDtypeStruct((NC, N, L), jnp.float32),
    mesh=_mesh,
    compiler_params=_sc_params,
    scratch_types=[
        pltpu.VMEM((CPW, CHUNK), jnp.int32),
        pltpu.VMEM((CHUNK, L), jnp.float32),
        pltpu.VMEM((ROWS_PT, L), jnp.float32),
        pltpu.SemaphoreType.DMA,
        pltpu.SemaphoreType.DMA,
        pltpu.SemaphoreType.DMA,
        pltpu.SemaphoreType.DMA,
        pltpu.VMEM_SHARED((N, L), jnp.float32),
    ],
)
def _sc_degree(dstR, out, dst_v, ones_v, zb, t0, t1, t2, t3, acc):
    cid = lax.axis_index("c")
    sid = lax.axis_index("s")
    wid = cid * NS + sid

    def fill(i, _):
        zb[i, :] = jnp.zeros((L,), jnp.float32)
        return 0

    lax.fori_loop(0, ROWS_PT, fill, 0, unroll=False)

    def fill1(i, _):
        ones_v[i, :] = jnp.ones((L,), jnp.float32)
        return 0

    lax.fori_loop(0, CHUNK, fill1, 0, unroll=False)
    pltpu.sync_copy(zb, acc.at[pl.ds(sid * ROWS_PT, ROWS_PT)])
    pltpu.sync_copy(dstR.at[wid], dst_v)
    plsc.subcore_barrier()

    ssems = (t0, t1, t2, t3)

    def group(i, _):
        for b in range(4):
            j = 4 * i + b

            @pl.when(j >= 4)
            def _():
                pltpu.make_async_copy(
                    ones_v, acc.at[dst_v.at[j - 4]], ssems[b]).wait()

            pltpu.async_copy(ones_v, acc.at[dst_v.at[j]], ssems[b])
        return 0

    lax.fori_loop(0, CPW // 4, group, 0, unroll=False)
    for b in range(4):
        j = CPW - 4 + b
        pltpu.make_async_copy(ones_v, acc.at[dst_v.at[j]], ssems[b]).wait()
    plsc.subcore_barrier()
    pltpu.sync_copy(acc.at[pl.ds(sid * ROWS_PT, ROWS_PT)],
                    out.at[cid, pl.ds(sid * ROWS_PT, ROWS_PT)])


@functools.partial(
    pl.kernel,
    out_type=jax.ShapeDtypeStruct((NC, N, L), jnp.float32),
    mesh=_mesh,
    compiler_params=_sc_params,
    scratch_types=[
        pltpu.VMEM((CPW, CHUNK), jnp.int32),
        pltpu.VMEM((CPW, CHUNK), jnp.int32),
        pltpu.VMEM((CHUNK, L), jnp.float32),
        pltpu.VMEM((CHUNK, L), jnp.float32),
        pltpu.VMEM((CHUNK, L), jnp.float32),
        pltpu.VMEM((CHUNK, L), jnp.float32),
        pltpu.VMEM((ROWS_PT, L), jnp.float32),
        pltpu.SemaphoreType.DMA,
        pltpu.SemaphoreType.DMA,
        pltpu.SemaphoreType.DMA,
        pltpu.SemaphoreType.DMA,
        pltpu.SemaphoreType.DMA,
        pltpu.SemaphoreType.DMA,
        pltpu.SemaphoreType.DMA,
        pltpu.SemaphoreType.DMA,
        pltpu.VMEM_SHARED((N, L), jnp.float32),
    ],
)
def _sc_spmm(tbl, srcR, dstR, out, src_v, dst_v, r0, r1, r2, r3, zb,
             s0, s1, s2, s3, t0, t1, t2, t3, acc):
    cid = lax.axis_index("c")
    sid = lax.axis_index("s")
    wid = cid * NS + sid

    def fill(i, _):
        zb[i, :] = jnp.zeros((L,), jnp.float32)
        return 0

    lax.fori_loop(0, ROWS_PT, fill, 0, unroll=False)
    pltpu.sync_copy(zb, acc.at[pl.ds(sid * ROWS_PT, ROWS_PT)])
    pltpu.sync_copy(srcR.at[wid], src_v)
    pltpu.sync_copy(dstR.at[wid], dst_v)
    plsc.subcore_barrier()

    rings = (r0, r1, r2, r3)
    gsems = (s0, s1, s2, s3)
    ssems = (t0, t1, t2, t3)
    for b in range(3):
        pltpu.async_copy(tbl.at[src_v.at[b]], rings[b], gsems[b])

    def group(i, _):
        for b in range(4):
            j = 4 * i + b
            pltpu.make_async_copy(tbl.at[src_v.at[j]], rings[b], gsems[b]).wait()
            pltpu.async_copy(rings[b], acc.at[dst_v.at[j]], ssems[b], add=True)
            nb = (b + 3) % 4

            @pl.when(j + 3 < CPW)
            def _():
                @pl.when(j >= 1)
                def _():
                    pltpu.make_async_copy(
                        rings[nb], acc.at[dst_v.at[j - 1]], ssems[nb]).wait()
                pltpu.async_copy(tbl.at[src_v.at[j + 3]], rings[nb], gsems[nb])
        return 0

    lax.fori_loop(0, CPW // 4, group, 0, unroll=False)
    for b in range(4):
        j = CPW - 4 + b
        pltpu.make_async_copy(rings[j % 4], acc.at[dst_v.at[j]],
                              ssems[j % 4]).wait()
    plsc.subcore_barrier()
    pltpu.sync_copy(acc.at[pl.ds(sid * ROWS_PT, ROWS_PT)],
                    out.at[cid, pl.ds(sid * ROWS_PT, ROWS_PT)])


def _tc_h1_body(x_ref, w_ref, d_ref, h_ref, dv_ref):
    deg = d_ref[0] + d_ref[1] + 1.0
    dinv = lax.rsqrt(deg)
    h = jnp.dot(x_ref[...], w_ref[...], preferred_element_type=jnp.float32)
    h_ref[...] = h * dinv
    dv_ref[...] = dinv


def _tc_mid_body(s_ref, h_ref, dv_ref, b_ref, g_ref):
    dinv = dv_ref[...]
    seg = s_ref[0] + s_ref[1] + h_ref[...]
    g_ref[...] = jnp.maximum(dinv * seg + b_ref[...], 0.0) * dinv


def _tc_out_body(a_ref, g_ref, dv_ref, w_ref, b_ref, o_ref):
    agg = a_ref[0] + a_ref[1] + g_ref[...]
    o = jnp.dot(agg, w_ref[...], preferred_element_type=jnp.float32)
    o_ref[...] = dv_ref[:, :D_OUT] * o + b_ref[...]


_tc_h1 = pl.pallas_call(
    _tc_h1_body,
    out_shape=(
        jax.ShapeDtypeStruct((N, L), jnp.float32),
        jax.ShapeDtypeStruct((N, L), jnp.float32),
    ),
)

_tc_mid = pl.pallas_call(
    _tc_mid_body,
    out_shape=jax.ShapeDtypeStruct((N, L), jnp.float32),
)

_tc_out = pl.pallas_call(
    _tc_out_body,
    out_shape=jax.ShapeDtypeStruct((N, D_OUT), jnp.float32),
)


def kernel(x, edge_index, W1, b1, W2, b2):
    srcR = edge_index[0].reshape(NW, CPW, CHUNK)
    dstR = edge_index[1].reshape(NW, CPW, CHUNK)

    degp = _sc_degree(dstR)
    h1, dinv16 = _tc_h1(x, W1, degp)
    seg1 = _sc_spmm(h1, srcR, dstR)
    g = _tc_mid(seg1, h1, dinv16, b1.reshape(1, L))
    agg = _sc_spmm(g, srcR, dstR)
    return _tc_out(agg, g, dinv16, W2, b2.reshape(1, D_OUT))
